# baseline (speedup 1.0000x reference)
import functools

import jax
import jax.numpy as jnp
from jax import lax
from jax.experimental import pallas as pl
from jax.experimental.pallas import tpu as pltpu
from jax.experimental.pallas import tpu_sc as plsc

BATCH = 4096
HIST = 200
EMBED_DIM = 64
SIZE = 128
NUM_EMB = 1000000

N = BATCH * HIST
GROW = 128
NW = 32
IDX_ROWS = N // GROW
ROWS_PER_W = IDX_ROWS // NW


def _tc_project_table(tableT, WT):
    BT = 16384

    def mm(t_ref, w_ref, p_ref):
        p_ref[...] = lax.dot_general(
            t_ref[...], w_ref[...],
            (((0,), (0,)), ((), ())),
            preferred_element_type=jnp.float32,
        )

    return pl.pallas_call(
        mm,
        grid=((NUM_EMB + BT - 1) // BT,),
        in_specs=[
            pl.BlockSpec((EMBED_DIM, BT), lambda i: (0, i)),
            pl.BlockSpec((EMBED_DIM, SIZE), lambda i: (0, 0)),
        ],
        out_specs=pl.BlockSpec((BT, SIZE), lambda i: (i, 0)),
        out_shape=jax.ShapeDtypeStruct((NUM_EMB, SIZE), jnp.float32),
    )(tableT, WT)


def _sc_gather(P, idx2d):
    mesh = plsc.VectorSubcoreMesh(
        core_axis_name="c", subcore_axis_name="s", num_cores=2, num_subcores=16
    )

    NBUF = 4
    NT = ROWS_PER_W // NBUF

    @functools.partial(
        pl.kernel,
        out_type=jax.ShapeDtypeStruct((N, SIZE), jnp.float32),
        mesh=mesh,
        scratch_types=[
            pltpu.VMEM((ROWS_PER_W, GROW), jnp.int32),
            [pltpu.VMEM((GROW, SIZE), jnp.float32) for _ in range(NBUF)],
            [pltpu.SemaphoreType.DMA for _ in range(NBUF)],
            [pltpu.SemaphoreType.DMA for _ in range(NBUF)],
        ],
        compiler_params=pltpu.CompilerParams(use_tc_tiling_on_sc=True),
    )
    def k(p_hbm, idx_hbm, out_hbm, idx_v, rows, gsem, wsem):
        wid = lax.axis_index("s") * 2 + lax.axis_index("c")
        base = wid * ROWS_PER_W
        pltpu.sync_copy(idx_hbm.at[pl.ds(base, ROWS_PER_W)], idx_v)

        for b in range(NBUF):
            pltpu.make_async_copy(p_hbm.at[idx_v.at[b]], rows[b], gsem[b]).start()

        def out_slot(j):
            return out_hbm.at[pl.ds((base + j) * GROW, GROW)]

        def body(t, carry):
            j0 = t * NBUF
            for b in range(NBUF):
                pltpu.make_async_copy(
                    p_hbm.at[idx_v.at[j0 + b]], rows[b], gsem[b]).wait()
                pltpu.make_async_copy(rows[b], out_slot(j0 + b), wsem[b]).start()

            @pl.when(t + 1 < NT)
            def _():
                for b in range(NBUF):
                    j2 = j0 + NBUF + b
                    pltpu.make_async_copy(rows[b], out_slot(j0 + b), wsem[b]).wait()
                    pltpu.make_async_copy(
                        p_hbm.at[idx_v.at[j2]], rows[b], gsem[b]).start()

            return carry

        lax.fori_loop(0, NT, body, 0)
        for b in range(NBUF):
            j = (NT - 1) * NBUF + b
            pltpu.make_async_copy(rows[b], out_slot(j), wsem[b]).wait()

    return k(P, idx2d)


def kernel(input, table, W):
    idx2d = input.reshape(IDX_ROWS, GROW).astype(jnp.int32)
    P = _tc_project_table(table.T, W.T)
    out = _sc_gather(P, idx2d)
    return out.reshape(BATCH, HIST, SIZE)
